# baseline (speedup 1.0000x reference)
import functools

import jax
import jax.numpy as jnp
import numpy as np
from jax import lax
from jax.experimental import pallas as pl
from jax.experimental.pallas import tpu as pltpu
from jax.experimental.pallas import tpu_sc as plsc

H = 8
F32 = jnp.float32
BF16 = jnp.bfloat16


def _dot(a, b):
    return jnp.dot(a, b, preferred_element_type=F32)


def _bdot(a, b):
    return jnp.dot(a.astype(BF16), b, preferred_element_type=F32)


def _sc_gather_call(tab, ia3, ib3, ic2):
    NW, NCH, CH = ia3.shape
    CW = ic2.shape[1]
    EW = NCH * CH
    E = NW * EW
    I32 = jnp.int32
    mesh = plsc.VectorSubcoreMesh(core_axis_name="c", subcore_axis_name="s")

    @functools.partial(
        pl.kernel, mesh=mesh,
        out_type=[jax.ShapeDtypeStruct((E, 128), I32),
                  jax.ShapeDtypeStruct((E, 128), I32),
                  jax.ShapeDtypeStruct((NW * CW, 128), I32)],
        scratch_types=[pltpu.VMEM((NCH, CH), I32),
                       pltpu.VMEM((NCH, CH), I32),
                       pltpu.VMEM((CW,), I32),
                       pltpu.VMEM((CH, 128), I32),
                       pltpu.VMEM((CH, 128), I32),
                       pltpu.VMEM((CH, 128), I32),
                       pltpu.VMEM((CH, 128), I32),
                       pltpu.VMEM((CW, 128), I32),
                       pltpu.SemaphoreType.DMA,
                       pltpu.SemaphoreType.DMA,
                       pltpu.SemaphoreType.DMA,
                       pltpu.SemaphoreType.DMA],
    )
    def k(tabr, ia, ib, ic, oa, ob, oc,
          ia_v, ib_v, ic_v, ra0, rb0, ra1, rb1, rc_v, s1, s2, s3, s4):
        wid = lax.axis_index("s") * 2 + lax.axis_index("c")
        base = wid * EW
        pltpu.sync_copy(ia.at[wid], ia_v)
        pltpu.sync_copy(ib.at[wid], ib_v)
        pltpu.sync_copy(ic.at[wid], ic_v)
        pltpu.async_copy(tabr.at[ic_v], rc_v, s1).wait()
        pltpu.sync_copy(rc_v, oc.at[pl.ds(wid * CW, CW)])

        def body(j, carry):
            ca = pltpu.async_copy(tabr.at[ia_v.at[2 * j]], ra0, s1)
            cb = pltpu.async_copy(tabr.at[ib_v.at[2 * j]], rb0, s2)
            cc = pltpu.async_copy(tabr.at[ia_v.at[2 * j + 1]], ra1, s3)
            cd = pltpu.async_copy(tabr.at[ib_v.at[2 * j + 1]], rb1, s4)
            ca.wait()
            pltpu.sync_copy(ra0, oa.at[pl.ds(base + 2 * j * CH, CH)])
            cb.wait()
            pltpu.sync_copy(rb0, ob.at[pl.ds(base + 2 * j * CH, CH)])
            cc.wait()
            pltpu.sync_copy(ra1, oa.at[pl.ds(base + (2 * j + 1) * CH, CH)])
            cd.wait()
            pltpu.sync_copy(rb1, ob.at[pl.ds(base + (2 * j + 1) * CH, CH)])
            return carry

        lax.fori_loop(0, NCH // 2, body, 0)

    return k(tab, ia3, ib3, ic2)


def _edge_body(xi_ref, xj_ref, eg_ref,
               We1a_ref, We1b_ref, We1c_ref, be1_ref, We2_ref, be2_ref,
               Wca_ref, Wcb_ref, bc_ref, A1_ref, ba1_ref, A2_ref, ba2_ref,
               Wv_ref, bv_ref, P_ref, GP_ref,
               trip_ref, prob_ref, val_ref):
    xi = xi_ref[...]
    xj = xj_ref[...]
    eg = eg_ref[...]
    pre = (_dot(xi, We1a_ref[...]) + _dot(eg, We1b_ref[...])
           + _dot(xj, We1c_ref[...]) + be1_ref[...])
    trip_ref[...] = _bdot(jax.nn.relu(pre), We2_ref[...]) + be2_ref[...]

    c = _dot(xi, Wca_ref[...]) + _dot(eg, Wcb_ref[...]) + bc_ref[...]
    hh = jax.nn.relu(_bdot(c, A1_ref[...]) + ba1_ref[...])
    att = (_bdot(hh, A2_ref[...]) + ba2_ref[...]) * (1.0 / np.sqrt(32.0))
    m = jnp.max(att, axis=1, keepdims=True)
    e = jnp.exp(att - m)
    e_flat = _bdot(e, P_ref[...])
    s_flat = _bdot(e, GP_ref[...])
    prob_flat = e_flat / s_flat
    prob_ref[...] = prob_flat
    v = _dot(xj, Wv_ref[...]) + bv_ref[...]
    val_ref[...] = prob_flat * v


def _edge_call(xi, xj, eg, We1a, We1b, We1c, be1, We2, be2,
               Wca, Wcb, bc, A1, ba1, A2, ba2, Wv, bv, P, GP, BE):
    E = eg.shape[0]
    DN = xi.shape[1]
    grid = E // BE
    row = lambda i: (i, 0)
    full = lambda i: (0, 0)
    bspec_e = pl.BlockSpec((BE, DN), row)
    wspec = lambda a: pl.BlockSpec(a.shape, full)
    return pl.pallas_call(
        _edge_body,
        interpret=False,
        grid=(grid,),
        in_specs=[bspec_e, bspec_e, bspec_e] + [wspec(a) for a in (
            We1a, We1b, We1c, be1, We2, be2, Wca, Wcb, bc, A1, ba1, A2, ba2,
            Wv, bv, P, GP)],
        out_specs=[bspec_e, bspec_e, bspec_e],
        out_shape=[jax.ShapeDtypeStruct((E, DN), F32),
                   jax.ShapeDtypeStruct((E, DN), F32),
                   jax.ShapeDtypeStruct((E, DN), F32)],
    )(xi, xj, eg, We1a, We1b, We1c, be1, We2, be2,
      Wca, Wcb, bc, A1, ba1, A2, ba2, Wv, bv, P, GP)


def _mv_body(xi2_ref, ids0_ref, image_ref,
             Wq2_ref, bq2_ref, Wk2_ref, bk2_ref, Wv2_ref, bv2_ref, y_ref):
    E2 = xi2_ref.shape[0]
    M = image_ref.shape[0]
    ids0 = ids0_ref[...]
    iota = jax.lax.broadcasted_iota(jnp.int32, (E2, M), 1)
    oh = (iota == ids0).astype(F32)
    xj2 = _dot(oh, image_ref[...])
    q2 = _dot(xi2_ref[...], Wq2_ref[...]) + bq2_ref[...]
    k2 = _dot(xj2, Wk2_ref[...]) + bk2_ref[...]
    v2 = _dot(xj2, Wv2_ref[...]) + bv2_ref[...]
    scale = 1.0 / np.sqrt(256.0)
    ys = []
    for h in range(H):
        qh = q2[:, 32 * h:32 * h + 32]
        kh = k2[:, 32 * h:32 * h + 32]
        vh = v2[:, 32 * h:32 * h + 32]
        s = jax.lax.dot_general(qh, kh, (((1,), (1,)), ((), ())),
                                preferred_element_type=F32) * scale
        m = jnp.max(s, axis=1, keepdims=True)
        e = jnp.exp(s - m)
        a = e / jnp.sum(e, axis=1, keepdims=True)
        ys.append(_dot(a, vh))
    y_ref[...] = jnp.concatenate(ys, axis=1)


def _mv_call(xi2, ids0, image, Wq2, bq2, Wk2, bk2, Wv2, bv2):
    E2, DN = xi2.shape
    args = (xi2, ids0, image, Wq2, bq2, Wk2, bk2, Wv2, bv2)
    return pl.pallas_call(
        _mv_body,
        interpret=False,
        in_specs=[pl.BlockSpec(a.shape, lambda: (0,) * 2) for a in args],
        out_specs=pl.BlockSpec((E2, DN), lambda: (0, 0)),
        out_shape=jax.ShapeDtypeStruct((E2, DN), F32),
    )(*args)


def _final_body(node_ref, agg_ref, y_ref, ids1_ref,
                Wu1a_ref, Wu1b_ref, bu1_ref, Wu2_ref, bu2_ref,
                Wnna_ref, Wnnb_ref, bnn_ref, out_ref, *, BN):
    i = pl.program_id(0)
    E2 = y_ref.shape[0]
    y16 = y_ref[...].astype(BF16)
    nf = jax.nn.relu(_bdot(node_ref[...], Wu1a_ref[...])
                     + _bdot(agg_ref[...], Wu1b_ref[...]) + bu1_ref[...])
    node_fan = _bdot(nf, Wu2_ref[...]) + bu2_ref[...]
    rowids = jax.lax.broadcasted_iota(jnp.int32, (BN, E2), 0) + i * BN
    oh = (rowids == ids1_ref[...]).astype(BF16)
    img = jnp.dot(oh, y16, preferred_element_type=F32)
    out_ref[...] = (_bdot(node_fan, Wnna_ref[...]) + _bdot(img, Wnnb_ref[...])
                    + bnn_ref[...])


def _final_call(node, agg, y, ids1, Wu1a, Wu1b, bu1, Wu2, bu2,
                Wnna, Wnnb, bnn, BN):
    N, DN = node.shape
    grid = N // BN
    row = lambda i: (i, 0)
    full = lambda i: (0, 0)
    nspec = pl.BlockSpec((BN, DN), row)
    args = (node, agg, y, ids1, Wu1a, Wu1b, bu1, Wu2, bu2, Wnna, Wnnb, bnn)
    return pl.pallas_call(
        functools.partial(_final_body, BN=BN),
        interpret=False,
        grid=(grid,),
        in_specs=[nspec, nspec] + [pl.BlockSpec(a.shape, full)
                                   for a in args[2:]],
        out_specs=nspec,
        out_shape=jax.ShapeDtypeStruct((N, DN), F32),
    )(*args)


def kernel(node, image, edge, edge_index_node_2_node, edge_index_image_2_ndoe,
           Wq, bq, Wk, bk, Wv, bv, We1, be1, We2, be2,
           Wa1, ba1, Wa2, ba2, Wu1, bu1, Wu2, bu2,
           Wq2, bq2, Wk2, bk2, Wv2, bv2, Wnn, bnn):
    N, DN = node.shape
    E = edge.shape[0]
    E2 = edge_index_image_2_ndoe.shape[1]
    ei = edge_index_node_2_node
    ei2 = edge_index_image_2_ndoe

    DH = DN // H
    to_hm = lambda W: W.reshape(-1, DH, H).transpose(0, 2, 1).reshape(-1, DN)
    Q_hm = to_hm(Wq)
    K_hm = to_hm(Wk)
    bq_hm = bq.reshape(DH, H).T.reshape(DN)
    bk_hm = bk.reshape(DH, H).T.reshape(DN)
    eyeN = jnp.eye(DN, dtype=F32)
    P = eyeN.reshape(DN, DH, H).transpose(0, 2, 1).reshape(DN, DN).T
    G = jnp.kron(jnp.eye(H, dtype=F32), jnp.ones((DH, DH), F32))
    GP = _dot(G, P)
    z = jnp.zeros((DN, H, DH), F32)
    Wca = jnp.concatenate([Q_hm.reshape(DN, H, DH), z], axis=2).reshape(DN, 2 * DN)
    Wcb = jnp.concatenate([z, K_hm.reshape(DN, H, DH)], axis=2).reshape(DN, 2 * DN)
    bc = jnp.concatenate([bq_hm.reshape(H, DH), bk_hm.reshape(H, DH)],
                         axis=1).reshape(2 * DN)
    A1 = jnp.kron(jnp.eye(H, dtype=F32), Wa1.T)
    A2 = jnp.kron(jnp.eye(H, dtype=F32), Wa2.T)
    ba1big = jnp.tile(ba1, H)
    ba2big = jnp.tile(ba2, H)
    We1a, We1b, We1c = We1[:DN], We1[DN:2 * DN], We1[2 * DN:]
    Wu1a, Wu1b = Wu1[:DN], Wu1[DN:]
    Wnna, Wnnb = Wnn[:DN], Wnn[DN:]
    r2 = lambda b: b.reshape(1, -1).astype(F32)
    b16 = lambda w: w.astype(BF16)

    node16 = node.astype(BF16)
    edge16 = edge.astype(BF16)
    nodei = lax.bitcast_convert_type(node16.reshape(N, DN // 2, 2), jnp.int32)
    NW, NCH, CH = 32, 40, 128
    Epad = NW * NCH * CH
    pad = lambda ix: jnp.pad(ix, (0, Epad - E)).reshape(NW, NCH, CH)
    ia3 = pad(ei[0])
    ib3 = pad(ei[1])
    ic2 = ei2[1].reshape(NW, E2 // NW)
    ga, gb, gc = _sc_gather_call(nodei, ia3, ib3, ic2)
    unbc = lambda g: lax.bitcast_convert_type(g, BF16).reshape(-1, DN)
    x_i = unbc(ga)
    x_j = unbc(gb)
    xi2 = unbc(gc).astype(F32)

    BE = 640 if E % 640 == 0 else E
    trip, prob_flat, value = _edge_call(
        x_i, x_j, edge16,
        b16(We1a), b16(We1b), b16(We1c), r2(be1), b16(We2), r2(be2),
        b16(Wca), b16(Wcb), r2(bc),
        b16(A1), r2(ba1big), b16(A2), r2(ba2big),
        b16(Wv), r2(bv), b16(P), b16(GP), BE)

    agg = jax.ops.segment_sum(value, ei[0], num_segments=N)

    y = _mv_call(xi2, ei2[0].reshape(E2, 1), image,
                 Wq2, r2(bq2), Wk2, r2(bk2), Wv2, r2(bv2))

    node_update = _final_call(node16, agg, y, ei2[1].reshape(1, E2),
                              b16(Wu1a), b16(Wu1b), r2(bu1), b16(Wu2), r2(bu2),
                              b16(Wnna), b16(Wnnb), r2(bnn),
                              BN=1000 if N % 1000 == 0 else N)

    return (node_update, trip, prob_flat.reshape(E, 32, 8))


# speedup vs baseline: 1.5863x; 1.5863x over previous
import functools

import jax
import jax.numpy as jnp
import numpy as np
from jax import lax
from jax.experimental import pallas as pl
from jax.experimental.pallas import tpu as pltpu
from jax.experimental.pallas import tpu_sc as plsc

H = 8
F32 = jnp.float32
BF16 = jnp.bfloat16


def _dot(a, b):
    return jnp.dot(a, b, preferred_element_type=F32)


def _bdot(a, b):
    return jnp.dot(a.astype(BF16), b, preferred_element_type=F32)


def _sc_gather_call(tab, ia3, ib3, ic2):
    NW, NCH, CH = ia3.shape
    CW = ic2.shape[1]
    EW = NCH * CH
    E = NW * EW
    I32 = jnp.int32
    mesh = plsc.VectorSubcoreMesh(core_axis_name="c", subcore_axis_name="s")

    @functools.partial(
        pl.kernel, mesh=mesh,
        out_type=[jax.ShapeDtypeStruct((E, 128), I32),
                  jax.ShapeDtypeStruct((E, 128), I32),
                  jax.ShapeDtypeStruct((NW * CW, 128), I32)],
        scratch_types=[pltpu.VMEM((NCH, CH), I32),
                       pltpu.VMEM((NCH, CH), I32),
                       pltpu.VMEM((CW,), I32),
                       pltpu.VMEM((CH, 128), I32),
                       pltpu.VMEM((CH, 128), I32),
                       pltpu.VMEM((CH, 128), I32),
                       pltpu.VMEM((CH, 128), I32),
                       pltpu.VMEM((CW, 128), I32),
                       pltpu.SemaphoreType.DMA,
                       pltpu.SemaphoreType.DMA,
                       pltpu.SemaphoreType.DMA,
                       pltpu.SemaphoreType.DMA],
    )
    def k(tabr, ia, ib, ic, oa, ob, oc,
          ia_v, ib_v, ic_v, ra0, rb0, ra1, rb1, rc_v, s1, s2, s3, s4):
        wid = lax.axis_index("s") * 2 + lax.axis_index("c")
        base = wid * EW
        pltpu.sync_copy(ia.at[wid], ia_v)
        pltpu.sync_copy(ib.at[wid], ib_v)
        pltpu.sync_copy(ic.at[wid], ic_v)
        pltpu.async_copy(tabr.at[ic_v], rc_v, s1).wait()
        pltpu.sync_copy(rc_v, oc.at[pl.ds(wid * CW, CW)])

        def body(j, carry):
            ca = pltpu.async_copy(tabr.at[ia_v.at[2 * j]], ra0, s1)
            cb = pltpu.async_copy(tabr.at[ib_v.at[2 * j]], rb0, s2)
            cc = pltpu.async_copy(tabr.at[ia_v.at[2 * j + 1]], ra1, s3)
            cd = pltpu.async_copy(tabr.at[ib_v.at[2 * j + 1]], rb1, s4)
            ca.wait()
            pltpu.sync_copy(ra0, oa.at[pl.ds(base + 2 * j * CH, CH)])
            cb.wait()
            pltpu.sync_copy(rb0, ob.at[pl.ds(base + 2 * j * CH, CH)])
            cc.wait()
            pltpu.sync_copy(ra1, oa.at[pl.ds(base + (2 * j + 1) * CH, CH)])
            cd.wait()
            pltpu.sync_copy(rb1, ob.at[pl.ds(base + (2 * j + 1) * CH, CH)])
            return carry

        lax.fori_loop(0, NCH // 2, body, 0)

    return k(tab, ia3, ib3, ic2)


def _unpack16(w_ref):
    w = w_ref[...]
    lo = lax.bitcast_convert_type(w << 16, F32).astype(BF16)
    hi = lax.bitcast_convert_type(w & jnp.int32(-65536), F32).astype(BF16)
    return lo, hi


def _edge_body(xi_ref, xj_ref, eg_ref,
               We1ae_ref, We1ao_ref, We1b_ref, We1ce_ref, We1co_ref,
               be1_ref, We2_ref, be2_ref,
               Wcae_ref, Wcao_ref, Wcb_ref, bc_ref,
               A1_ref, ba1_ref, A2_ref, ba2_ref,
               Wve_ref, Wvo_ref, bv_ref, P_ref, GP_ref,
               trip_ref, prob_ref, val_ref):
    xie, xio = _unpack16(xi_ref)
    xje, xjo = _unpack16(xj_ref)
    eg = eg_ref[...].astype(BF16)
    pre = (_dot(xie, We1ae_ref[...]) + _dot(xio, We1ao_ref[...])
           + _dot(eg, We1b_ref[...])
           + _dot(xje, We1ce_ref[...]) + _dot(xjo, We1co_ref[...])
           + be1_ref[...])
    trip_ref[...] = _bdot(jax.nn.relu(pre), We2_ref[...]) + be2_ref[...]

    c = (_dot(xie, Wcae_ref[...]) + _dot(xio, Wcao_ref[...])
         + _dot(eg, Wcb_ref[...]) + bc_ref[...])
    hh = jax.nn.relu(_bdot(c, A1_ref[...]) + ba1_ref[...])
    att = (_bdot(hh, A2_ref[...]) + ba2_ref[...]) * (1.0 / np.sqrt(32.0))
    m = jnp.max(att, axis=1, keepdims=True)
    e = jnp.exp(att - m)
    e_flat = _bdot(e, P_ref[...])
    s_flat = _bdot(e, GP_ref[...])
    prob_flat = e_flat / s_flat
    prob_ref[...] = prob_flat
    v = (_dot(xje, Wve_ref[...]) + _dot(xjo, Wvo_ref[...])
         + bv_ref[...])
    val_ref[...] = prob_flat * v


def _edge_call(xi, xj, eg, *ws, BE):
    E, DN = eg.shape
    grid = E // BE
    row = lambda i: (i, 0)
    full = lambda i: (0, 0)
    bspec_e = pl.BlockSpec((BE, DN), row)
    bspec_p = pl.BlockSpec((BE, DN // 2), row)
    wspec = lambda a: pl.BlockSpec(a.shape, full)
    return pl.pallas_call(
        _edge_body,
        interpret=False,
        grid=(grid,),
        in_specs=[bspec_p, bspec_p, bspec_e] + [wspec(a) for a in ws],
        out_specs=[bspec_e, bspec_e, bspec_e],
        out_shape=[jax.ShapeDtypeStruct((E, DN), F32),
                   jax.ShapeDtypeStruct((E, DN), F32),
                   jax.ShapeDtypeStruct((E, DN), F32)],
    )(xi, xj, eg, *ws)


def _mv_body(xi2_ref, ids0_ref, image_ref,
             Wq2e_ref, Wq2o_ref, bq2_ref, Wk2_ref, bk2_ref, Wv2_ref,
             bv2_ref, y_ref):
    E2 = xi2_ref.shape[0]
    M = image_ref.shape[0]
    ids0 = ids0_ref[...]
    iota = jax.lax.broadcasted_iota(jnp.int32, (E2, M), 1)
    oh = (iota == ids0).astype(F32)
    xj2 = _dot(oh, image_ref[...])
    x2e, x2o = _unpack16(xi2_ref)
    q2 = (_dot(x2e.astype(F32), Wq2e_ref[...])
          + _dot(x2o.astype(F32), Wq2o_ref[...]) + bq2_ref[...])
    k2 = _dot(xj2, Wk2_ref[...]) + bk2_ref[...]
    v2 = _dot(xj2, Wv2_ref[...]) + bv2_ref[...]
    scale = 1.0 / np.sqrt(256.0)
    ys = []
    for h in range(H):
        qh = q2[:, 32 * h:32 * h + 32]
        kh = k2[:, 32 * h:32 * h + 32]
        vh = v2[:, 32 * h:32 * h + 32]
        s = jax.lax.dot_general(qh, kh, (((1,), (1,)), ((), ())),
                                preferred_element_type=F32) * scale
        m = jnp.max(s, axis=1, keepdims=True)
        e = jnp.exp(s - m)
        a = e / jnp.sum(e, axis=1, keepdims=True)
        ys.append(_dot(a, vh))
    y_ref[...] = jnp.concatenate(ys, axis=1)


def _mv_call(xi2, ids0, image, Wq2e, Wq2o, bq2, Wk2, bk2, Wv2, bv2):
    E2 = xi2.shape[0]
    DN = image.shape[1]
    args = (xi2, ids0, image, Wq2e, Wq2o, bq2, Wk2, bk2, Wv2, bv2)
    return pl.pallas_call(
        _mv_body,
        interpret=False,
        in_specs=[pl.BlockSpec(a.shape, lambda: (0,) * 2) for a in args],
        out_specs=pl.BlockSpec((E2, DN), lambda: (0, 0)),
        out_shape=jax.ShapeDtypeStruct((E2, DN), F32),
    )(*args)


def _final_body(node_ref, agg_ref, y_ref, ids1_ref,
                Wu1a_ref, Wu1b_ref, bu1_ref, Wu2_ref, bu2_ref,
                Wnna_ref, Wnnb_ref, bnn_ref, out_ref, *, BN):
    i = pl.program_id(0)
    E2 = y_ref.shape[0]
    y16 = y_ref[...].astype(BF16)
    nf = jax.nn.relu(_bdot(node_ref[...], Wu1a_ref[...])
                     + _bdot(agg_ref[...], Wu1b_ref[...]) + bu1_ref[...])
    node_fan = _bdot(nf, Wu2_ref[...]) + bu2_ref[...]
    rowids = jax.lax.broadcasted_iota(jnp.int32, (BN, E2), 0) + i * BN
    oh = (rowids == ids1_ref[...]).astype(BF16)
    img = jnp.dot(oh, y16, preferred_element_type=F32)
    out_ref[...] = (_bdot(node_fan, Wnna_ref[...]) + _bdot(img, Wnnb_ref[...])
                    + bnn_ref[...])


def _final_call(node, agg, y, ids1, Wu1a, Wu1b, bu1, Wu2, bu2,
                Wnna, Wnnb, bnn, BN):
    N, DN = node.shape
    grid = N // BN
    row = lambda i: (i, 0)
    full = lambda i: (0, 0)
    nspec = pl.BlockSpec((BN, DN), row)
    args = (node, agg, y, ids1, Wu1a, Wu1b, bu1, Wu2, bu2, Wnna, Wnnb, bnn)
    return pl.pallas_call(
        functools.partial(_final_body, BN=BN),
        interpret=False,
        grid=(grid,),
        in_specs=[nspec, nspec] + [pl.BlockSpec(a.shape, full)
                                   for a in args[2:]],
        out_specs=nspec,
        out_shape=jax.ShapeDtypeStruct((N, DN), F32),
    )(*args)


def kernel(node, image, edge, edge_index_node_2_node, edge_index_image_2_ndoe,
           Wq, bq, Wk, bk, Wv, bv, We1, be1, We2, be2,
           Wa1, ba1, Wa2, ba2, Wu1, bu1, Wu2, bu2,
           Wq2, bq2, Wk2, bk2, Wv2, bv2, Wnn, bnn):
    N, DN = node.shape
    E = edge.shape[0]
    E2 = edge_index_image_2_ndoe.shape[1]
    ei = edge_index_node_2_node
    ei2 = edge_index_image_2_ndoe

    DH = DN // H
    to_hm = lambda W: W.reshape(-1, DH, H).transpose(0, 2, 1).reshape(-1, DN)
    Q_hm = to_hm(Wq)
    K_hm = to_hm(Wk)
    bq_hm = bq.reshape(DH, H).T.reshape(DN)
    bk_hm = bk.reshape(DH, H).T.reshape(DN)
    eyeN = jnp.eye(DN, dtype=F32)
    P = eyeN.reshape(DN, DH, H).transpose(0, 2, 1).reshape(DN, DN).T
    G = jnp.kron(jnp.eye(H, dtype=F32), jnp.ones((DH, DH), F32))
    GP = _dot(G, P)
    z = jnp.zeros((DN, H, DH), F32)
    Wca = jnp.concatenate([Q_hm.reshape(DN, H, DH), z], axis=2).reshape(DN, 2 * DN)
    Wcb = jnp.concatenate([z, K_hm.reshape(DN, H, DH)], axis=2).reshape(DN, 2 * DN)
    bc = jnp.concatenate([bq_hm.reshape(H, DH), bk_hm.reshape(H, DH)],
                         axis=1).reshape(2 * DN)
    A1 = jnp.kron(jnp.eye(H, dtype=F32), Wa1.T)
    A2 = jnp.kron(jnp.eye(H, dtype=F32), Wa2.T)
    ba1big = jnp.tile(ba1, H)
    ba2big = jnp.tile(ba2, H)
    We1a, We1b, We1c = We1[:DN], We1[DN:2 * DN], We1[2 * DN:]
    Wu1a, Wu1b = Wu1[:DN], Wu1[DN:]
    Wnna, Wnnb = Wnn[:DN], Wnn[DN:]
    r2 = lambda b: b.reshape(1, -1).astype(F32)
    b16 = lambda w: w.astype(BF16)

    node16 = node.astype(BF16)
    nodei = lax.bitcast_convert_type(node16.reshape(N, DN // 2, 2), jnp.int32)
    NW, NCH, CH = 32, 40, 128
    Epad = NW * NCH * CH
    pad = lambda ix: jnp.pad(ix, (0, Epad - E)).reshape(NW, NCH, CH)
    ia3 = pad(ei[0])
    ib3 = pad(ei[1])
    ic2 = ei2[1].reshape(NW, E2 // NW)
    ga, gb, gc = _sc_gather_call(nodei, ia3, ib3, ic2)

    ev = lambda w: b16(w[0::2])
    od = lambda w: b16(w[1::2])
    BE = 640 if E % 640 == 0 else E
    trip, prob_flat, value = _edge_call(
        ga, gb, edge,
        ev(We1a), od(We1a), b16(We1b), ev(We1c), od(We1c), r2(be1),
        b16(We2), r2(be2),
        ev(Wca), od(Wca), b16(Wcb), r2(bc),
        b16(A1), r2(ba1big), b16(A2), r2(ba2big),
        ev(Wv), od(Wv), r2(bv), b16(P), b16(GP), BE=BE)

    agg = jax.ops.segment_sum(value, ei[0], num_segments=N)

    y = _mv_call(gc, ei2[0].reshape(E2, 1), image,
                 Wq2[0::2], Wq2[1::2], r2(bq2), Wk2, r2(bk2), Wv2, r2(bv2))

    node_update = _final_call(node16, agg, y, ei2[1].reshape(1, E2),
                              b16(Wu1a), b16(Wu1b), r2(bu1), b16(Wu2), r2(bu2),
                              b16(Wnna), b16(Wnnb), r2(bnn),
                              BN=1000 if N % 1000 == 0 else N)

    return (node_update, trip, prob_flat.reshape(E, 32, 8))
